# Initial kernel scaffold; baseline (speedup 1.0000x reference)
import jax
import jax.numpy as jnp
from jax.experimental import pallas as pl


def kernel(x, W0, b0, g0, be0, W1, b1, g1, be1, W2, b2, g2, be2, Wfc1, bfc1, gfc1, befc1, Wfc2, bfc2, gfc2, befc2, Wsm, bsm):
    raise NotImplementedError("write your pallas kernel here")


# speedup vs baseline: 1.0004x; 1.0004x over previous
import functools

import jax
import jax.numpy as jnp
import numpy as np
from jax.experimental import pallas as pl

_NUM_POINTS = [1024, 256, 64, 16]
_RADIUS = [0.2, 0.4, 0.8]
_PATCH = 32
_SHELLS = 3
_GAUSS_SCALE = 0.69314718056 * _SHELLS ** 2
_BN_EPS = 1e-3
_BN_SCALE = 1.0 / np.sqrt(1.0 + _BN_EPS)


def _kdtree_indexing(x):
    B, N, _ = x.shape
    depth = int(np.log2(N))
    y = x
    for lvl in range(depth):
        nb = 2 ** lvl
        blk = N // nb
        yb = y.reshape(B, nb, blk, 3)
        dim = lvl % 3
        order = jnp.argsort(yb[..., dim], axis=-1)
        yb = jnp.take_along_axis(yb, order[..., None], axis=2)
        y = yb.reshape(B, N, 3)
    return y


def _kd_pool(x, p):
    B, N, C = x.shape
    return x.reshape(B, N // p, p, C).mean(axis=2)


def _batch_gather(x, idx):
    return jax.vmap(lambda xi, ii: xi[ii])(x, idx)


def _group_points(source, target, radius, patch_size):
    d2 = jnp.sum((target[:, :, None, :] - source[:, None, :, :]) ** 2, axis=-1)
    _, idx = jax.lax.top_k(-d2, patch_size)
    patches = _batch_gather(source, idx)
    rel = (patches - target[:, :, None, :]) / radius
    dist = jnp.sqrt(jnp.maximum(jnp.sum(rel ** 2, axis=-1), 1e-12))
    return idx, rel, dist


def _real_sh(d):
    x = d[..., 0]; y = d[..., 1]; z = d[..., 2]
    sh = [0.28209479 * jnp.ones_like(x),
          0.48860251 * y, 0.48860251 * z, 0.48860251 * x,
          1.09254843 * x * y, 1.09254843 * y * z,
          0.31539157 * (3.0 * z * z - 1.0),
          1.09254843 * x * z, 0.54627421 * (x * x - y * y),
          0.59004359 * y * (3.0 * x * x - y * y),
          2.89061144 * x * y * z,
          0.45704579 * y * (5.0 * z * z - 1.0),
          0.37317633 * z * (5.0 * z * z - 3.0),
          0.45704579 * x * (5.0 * z * z - 1.0),
          1.44530572 * z * (x * x - y * y),
          0.59004359 * x * (x * x - 3.0 * y * y)]
    return jnp.stack(sh, axis=-1)


def _sh_gauss_kernel(rel, dist):
    direction = rel / jnp.maximum(dist[..., None], 1e-8)
    sh = _real_sh(direction)
    centers = jnp.arange(_SHELLS, dtype=jnp.float32) / (_SHELLS - 1)
    gauss = jnp.exp(-_GAUSS_SCALE * (dist[..., None] - centers) ** 2)
    gauss = gauss * (dist[..., None] <= 1.0).astype(jnp.float32)
    k = gauss[..., :, None] * sh[..., None, :]
    return k.reshape(k.shape[:-2] + (_SHELLS * 16,))


def _bn_relu(y, gamma, beta):
    return jax.nn.relu(gamma * (y * _BN_SCALE) + beta)


def _fc_head_body(y_ref, w1_ref, b1_ref, g1_ref, be1_ref,
                  w2_ref, b2_ref, g2_ref, be2_ref,
                  wsm_ref, bsm_ref, out_ref):
    y = jnp.max(y_ref[...], axis=1)
    h = jnp.dot(y, w1_ref[...], preferred_element_type=jnp.float32)
    h = (h + b1_ref[...]) * _BN_SCALE
    h = jax.nn.relu(g1_ref[...] * h + be1_ref[...])
    h = jnp.dot(h, w2_ref[...], preferred_element_type=jnp.float32)
    h = (h + b2_ref[...]) * _BN_SCALE
    h = jax.nn.relu(g2_ref[...] * h + be2_ref[...])
    logits = jnp.dot(h, wsm_ref[...], preferred_element_type=jnp.float32)
    logits = logits + bsm_ref[...]
    m = jnp.max(logits, axis=-1, keepdims=True)
    e = jnp.exp(logits - m)
    out_ref[...] = e / jnp.sum(e, axis=-1, keepdims=True)


def _fc_head(y, Wfc1, bfc1, gfc1, befc1, Wfc2, bfc2, gfc2, befc2, Wsm, bsm):
    B = y.shape[0]
    return pl.pallas_call(
        _fc_head_body,
        out_shape=jax.ShapeDtypeStruct((B, Wsm.shape[1]), jnp.float32),
    )(y, Wfc1, bfc1, gfc1, befc1, Wfc2, bfc2, gfc2, befc2, Wsm, bsm)


def kernel(x, W0, b0, g0, be0, W1, b1, g1, be1, W2, b2, g2, be2,
           Wfc1, bfc1, gfc1, befc1, Wfc2, bfc2, gfc2, befc2, Wsm, bsm):
    Ws = [W0, W1, W2]; bs = [b0, b1, b2]; gs = [g0, g1, g2]; bes = [be0, be1, be2]
    x = _kdtree_indexing(x)
    points = [x]
    for i in range(3):
        points.append(_kd_pool(points[-1], _NUM_POINTS[i] // _NUM_POINTS[i + 1]))
    yzx = [jnp.stack([p[..., 1], p[..., 2], p[..., 0]], axis=-1) for p in points]
    B = x.shape[0]
    y = jnp.ones((B, x.shape[1], 1), dtype=jnp.float32)
    for i in range(3):
        idx, rel, dist = _group_points(points[i], points[i + 1], _RADIUS[i], _PATCH)
        kmat = _sh_gauss_kernel(rel, dist)
        y = jnp.concatenate([y, yzx[i]], axis=-1)
        yp = _batch_gather(y, idx)
        y = jnp.einsum('bvpc,bvpd->bvcd', yp, kmat)
        y = y.reshape(B, y.shape[1], -1)
        y = _bn_relu(y @ Ws[i] + bs[i], gs[i], bes[i])
    return _fc_head(y, Wfc1, bfc1, gfc1, befc1, Wfc2, bfc2, gfc2, befc2, Wsm, bsm)


# speedup vs baseline: 3.6142x; 3.6127x over previous
import functools

import jax
import jax.numpy as jnp
import numpy as np
from jax.experimental import pallas as pl
from jax.experimental.pallas import tpu as pltpu

_NUM_POINTS = [1024, 256, 64, 16]
_RADIUS = [0.2, 0.4, 0.8]
_PATCH = 32
_SHELLS = 3
_GAUSS_SCALE = 0.69314718056 * _SHELLS ** 2
_BN_EPS = 1e-3
_BN_SCALE = 1.0 / np.sqrt(1.0 + _BN_EPS)


def _kdtree_indexing(x):
    B, N, _ = x.shape
    depth = int(np.log2(N))
    y = x
    for lvl in range(depth):
        nb = 2 ** lvl
        blk = N // nb
        yb = y.reshape(B, nb, blk, 3)
        dim = lvl % 3
        order = jnp.argsort(yb[..., dim], axis=-1)
        yb = jnp.take_along_axis(yb, order[..., None], axis=2)
        y = yb.reshape(B, N, 3)
    return y


def _kd_pool(x, p):
    B, N, C = x.shape
    return x.reshape(B, N // p, p, C).mean(axis=2)


def _sh_rows(dx, dy, dz):
    one = jnp.ones_like(dx)
    return [0.28209479 * one,
            0.48860251 * dy, 0.48860251 * dz, 0.48860251 * dx,
            1.09254843 * dx * dy, 1.09254843 * dy * dz,
            0.31539157 * (3.0 * dz * dz - 1.0),
            1.09254843 * dx * dz, 0.54627421 * (dx * dx - dy * dy),
            0.59004359 * dy * (3.0 * dx * dx - dy * dy),
            2.89061144 * dx * dy * dz,
            0.45704579 * dy * (5.0 * dz * dz - 1.0),
            0.37317633 * dz * (5.0 * dz * dz - 3.0),
            0.45704579 * dx * (5.0 * dz * dz - 1.0),
            1.44530572 * dz * (dx * dx - dy * dy),
            0.59004359 * dx * (dx * dx - 3.0 * dy * dy)]


def _make_level_body(radius, n_feat):
    inv_r = 1.0 / radius

    def body(*refs):
        if n_feat:
            s_ref, st_ref, tt_ref, ft_ref, e_ref = refs
        else:
            s_ref, st_ref, tt_ref, e_ref = refs
        s = s_ref[0]
        st = st_ref[0]
        tt = tt_ref[0]
        if n_feat:
            gat = jnp.concatenate([st, ft_ref[0]], axis=0)
        else:
            gat = st
        nt = tt.shape[1]
        c_dim = (n_feat + 3) if n_feat else 4

        ssq = jnp.sum(s * s, axis=1, keepdims=True)
        tsq = jnp.sum(tt * tt, axis=0, keepdims=True)
        cross = jnp.dot(s, tt, preferred_element_type=jnp.float32)
        d2t = ssq - 2.0 * cross + tsq

        e_init = jnp.zeros((c_dim * 48, nt), jnp.float32)

        def step(_, carry):
            d2, e = carry
            m = jnp.min(d2, axis=0, keepdims=True)
            sel = d2 <= m
            d2 = jnp.where(sel, 1e30, d2)
            oh = sel.astype(jnp.float32)
            g = jnp.dot(gat, oh, preferred_element_type=jnp.float32)
            gc = g[0:3]
            rel = (gc - tt) * inv_r
            dd = jnp.sum(rel * rel, axis=0, keepdims=True)
            dist = jnp.sqrt(jnp.maximum(dd, 1e-12))
            dirv = rel / jnp.maximum(dist, 1e-8)
            sh = jnp.concatenate(
                _sh_rows(dirv[0:1], dirv[1:2], dirv[2:3]), axis=0)
            in_r = (dist <= 1.0).astype(jnp.float32)
            shells = []
            for j in range(_SHELLS):
                cj = j / (_SHELLS - 1.0)
                gj = jnp.exp(-_GAUSS_SCALE * (dist - cj) ** 2) * in_r
                shells.append(gj * sh)
            k48 = jnp.concatenate(shells, axis=0)
            if n_feat:
                yp = jnp.concatenate(
                    [g[3:], gc[1:2], gc[2:3], gc[0:1]], axis=0)
            else:
                yp = jnp.concatenate(
                    [jnp.ones((1, nt), jnp.float32),
                     gc[1:2], gc[2:3], gc[0:1]], axis=0)
            yp3 = jnp.broadcast_to(yp[:, None, :], (c_dim, 48, nt))
            k3 = jnp.broadcast_to(k48[None, :, :], (c_dim, 48, nt))
            e = e + (yp3 * k3).reshape(c_dim * 48, nt)
            return d2, e

        _, e_out = jax.lax.fori_loop(0, _PATCH, step, (d2t, e_init))
        e_ref[0] = e_out

    return body


def _level_einsum(s, st, tt, ft, radius):
    B, Ns, _ = s.shape
    Nt = tt.shape[2]
    n_feat = 0 if ft is None else ft.shape[1]
    c_dim = (n_feat + 3) if n_feat else 4
    body = _make_level_body(radius, n_feat)
    in_specs = [
        pl.BlockSpec((1, Ns, 3), lambda b: (b, 0, 0)),
        pl.BlockSpec((1, 3, Ns), lambda b: (b, 0, 0)),
        pl.BlockSpec((1, 3, Nt), lambda b: (b, 0, 0)),
    ]
    args = [s, st, tt]
    if ft is not None:
        in_specs.append(pl.BlockSpec((1, n_feat, Ns), lambda b: (b, 0, 0)))
        args.append(ft)
    return pl.pallas_call(
        body,
        grid=(B,),
        in_specs=in_specs,
        out_specs=pl.BlockSpec((1, c_dim * 48, Nt), lambda b: (b, 0, 0)),
        out_shape=jax.ShapeDtypeStruct((B, c_dim * 48, Nt), jnp.float32),
        compiler_params=pltpu.CompilerParams(
            dimension_semantics=("parallel",)),
    )(*args)


def _mm_body(nk, wt_ref, xt_ref, b_ref, g_ref, be_ref, o_ref):
    k = pl.program_id(0)

    @pl.when(k == 0)
    def _():
        o_ref[...] = jnp.zeros_like(o_ref)

    o_ref[...] += jnp.dot(wt_ref[...], xt_ref[...],
                          preferred_element_type=jnp.float32)

    @pl.when(k == nk - 1)
    def _():
        acc = (o_ref[...] + b_ref[...]) * _BN_SCALE
        o_ref[...] = jax.nn.relu(g_ref[...] * acc + be_ref[...])


def _mlp_t(wt, xt, b, g, be, k_tile):
    U, K = wt.shape
    N = xt.shape[1]
    nk = K // k_tile
    assert K % k_tile == 0
    return pl.pallas_call(
        functools.partial(_mm_body, nk),
        grid=(nk,),
        in_specs=[
            pl.BlockSpec((U, k_tile), lambda k: (0, k)),
            pl.BlockSpec((k_tile, N), lambda k: (k, 0)),
            pl.BlockSpec((U, 1), lambda k: (0, 0)),
            pl.BlockSpec((U, 1), lambda k: (0, 0)),
            pl.BlockSpec((U, 1), lambda k: (0, 0)),
        ],
        out_specs=pl.BlockSpec((U, N), lambda k: (0, 0)),
        out_shape=jax.ShapeDtypeStruct((U, N), jnp.float32),
    )(wt, xt, b[:, None], g[:, None], be[:, None])


def _head_body(y_ref, w1_ref, b1_ref, g1_ref, be1_ref,
               w2_ref, b2_ref, g2_ref, be2_ref,
               wsm_ref, bsm_ref, o_ref):
    ym = jnp.max(y_ref[...], axis=1)
    h = jnp.dot(w1_ref[...], ym, preferred_element_type=jnp.float32)
    h = (h + b1_ref[...]) * _BN_SCALE
    h = jax.nn.relu(g1_ref[...] * h + be1_ref[...])
    h = jnp.dot(w2_ref[...], h, preferred_element_type=jnp.float32)
    h = (h + b2_ref[...]) * _BN_SCALE
    h = jax.nn.relu(g2_ref[...] * h + be2_ref[...])
    lg = jnp.dot(wsm_ref[...], h, preferred_element_type=jnp.float32)
    lg = lg + bsm_ref[...]
    m = jnp.max(lg, axis=0, keepdims=True)
    e = jnp.exp(lg - m)
    o_ref[...] = e / jnp.sum(e, axis=0, keepdims=True)


def kernel(x, W0, b0, g0, be0, W1, b1, g1, be1, W2, b2, g2, be2,
           Wfc1, bfc1, gfc1, befc1, Wfc2, bfc2, gfc2, befc2, Wsm, bsm):
    Ws = [W0, W1, W2]; bs = [b0, b1, b2]; gs = [g0, g1, g2]; bes = [be0, be1, be2]
    k_tiles = [192, 1664, 1792]
    k_pads = [192, 3328, 12544]
    x = _kdtree_indexing(x)
    points = [x]
    for i in range(3):
        points.append(_kd_pool(points[-1], _NUM_POINTS[i] // _NUM_POINTS[i + 1]))
    pt = [jnp.transpose(p, (0, 2, 1)) for p in points]
    B = x.shape[0]
    ft = None
    for i in range(3):
        Nt = _NUM_POINTS[i + 1]
        e = _level_einsum(points[i], pt[i], pt[i + 1], ft, _RADIUS[i])
        cd = e.shape[1]
        e_all = jnp.transpose(e, (1, 0, 2)).reshape(cd, B * Nt)
        kp = k_pads[i]
        wt = Ws[i].T
        if kp != cd:
            e_all = jnp.pad(e_all, ((0, kp - cd), (0, 0)))
            wt = jnp.pad(wt, ((0, 0), (0, kp - cd)))
        y = _mlp_t(wt, e_all, bs[i], gs[i], bes[i], k_tiles[i])
        U = y.shape[0]
        ft = jnp.transpose(y.reshape(U, B, Nt), (1, 0, 2))
    y3 = jnp.transpose(y.reshape(1024, B, 16), (0, 2, 1))
    out = pl.pallas_call(
        _head_body,
        out_shape=jax.ShapeDtypeStruct((Wsm.shape[1], B), jnp.float32),
    )(y3, Wfc1.T, bfc1[:, None], gfc1[:, None], befc1[:, None],
      Wfc2.T, bfc2[:, None], gfc2[:, None], befc2[:, None],
      Wsm.T, bsm[:, None])
    return out.T


# speedup vs baseline: 5.7467x; 1.5900x over previous
import functools

import jax
import jax.numpy as jnp
import numpy as np
from jax.experimental import pallas as pl
from jax.experimental.pallas import tpu as pltpu

_NUM_POINTS = [1024, 256, 64, 16]
_RADIUS = [0.2, 0.4, 0.8]
_PATCH = 32
_SHELLS = 3
_GAUSS_SCALE = 0.69314718056 * _SHELLS ** 2
_BN_EPS = 1e-3
_BN_SCALE = 1.0 / np.sqrt(1.0 + _BN_EPS)


def _kdtree_indexing(x):
    B, N, _ = x.shape
    depth = int(np.log2(N))
    y = x
    for lvl in range(depth):
        nb = 2 ** lvl
        blk = N // nb
        yb = y.reshape(B, nb, blk, 3)
        dim = lvl % 3
        order = jnp.argsort(yb[..., dim], axis=-1)
        yb = jnp.take_along_axis(yb, order[..., None], axis=2)
        y = yb.reshape(B, N, 3)
    return y


def _kd_pool(x, p):
    B, N, C = x.shape
    return x.reshape(B, N // p, p, C).mean(axis=2)


def _sh_rows(dx, dy, dz):
    one = jnp.ones_like(dx)
    return [0.28209479 * one,
            0.48860251 * dy, 0.48860251 * dz, 0.48860251 * dx,
            1.09254843 * dx * dy, 1.09254843 * dy * dz,
            0.31539157 * (3.0 * dz * dz - 1.0),
            1.09254843 * dx * dz, 0.54627421 * (dx * dx - dy * dy),
            0.59004359 * dy * (3.0 * dx * dx - dy * dy),
            2.89061144 * dx * dy * dz,
            0.45704579 * dy * (5.0 * dz * dz - 1.0),
            0.37317633 * dz * (5.0 * dz * dz - 3.0),
            0.45704579 * dx * (5.0 * dz * dz - 1.0),
            1.44530572 * dz * (dx * dx - dy * dy),
            0.59004359 * dx * (dx * dx - 3.0 * dy * dy)]


def _make_level_body(radius, n_feat):
    inv_r = 1.0 / radius

    def body(*refs):
        if n_feat:
            s_ref, st_ref, tt_ref, ft_ref, e_ref = refs
        else:
            s_ref, st_ref, tt_ref, e_ref = refs
        s = s_ref[0]
        st = st_ref[0]
        tt = tt_ref[0]
        if n_feat:
            gat = jnp.concatenate([st, ft_ref[0]], axis=0)
        else:
            gat = st
        nt = tt.shape[1]
        c_dim = (n_feat + 3) if n_feat else 4

        ssq = jnp.sum(s * s, axis=1, keepdims=True)
        tsq = jnp.sum(tt * tt, axis=0, keepdims=True)
        cross = jnp.dot(s, tt, preferred_element_type=jnp.float32)
        d2t = ssq - 2.0 * cross + tsq

        e_init = jnp.zeros((c_dim * 48, nt), jnp.float32)

        def step(_, carry):
            d2, e = carry
            m = jnp.min(d2, axis=0, keepdims=True)
            sel = d2 <= m
            d2 = jnp.where(sel, 1e30, d2)
            oh = sel.astype(jnp.float32)
            if n_feat:
                g = jnp.dot(gat, oh, preferred_element_type=jnp.float32)
                gc = g[0:3]
            else:
                gc = jnp.concatenate(
                    [jnp.sum(oh * s[:, r:r + 1], axis=0, keepdims=True)
                     for r in range(3)], axis=0)
            rel = (gc - tt) * inv_r
            dd = jnp.sum(rel * rel, axis=0, keepdims=True)
            dist = jnp.sqrt(jnp.maximum(dd, 1e-12))
            dirv = rel / jnp.maximum(dist, 1e-8)
            sh = jnp.concatenate(
                _sh_rows(dirv[0:1], dirv[1:2], dirv[2:3]), axis=0)
            in_r = (dist <= 1.0).astype(jnp.float32)
            shells = []
            for j in range(_SHELLS):
                cj = j / (_SHELLS - 1.0)
                gj = jnp.exp(-_GAUSS_SCALE * (dist - cj) ** 2) * in_r
                shells.append(gj * sh)
            k48 = jnp.concatenate(shells, axis=0)
            if n_feat:
                yp = jnp.concatenate(
                    [g[3:], gc[1:2], gc[2:3], gc[0:1]], axis=0)
            else:
                yp = jnp.concatenate(
                    [jnp.ones((1, nt), jnp.float32),
                     gc[1:2], gc[2:3], gc[0:1]], axis=0)
            yp3 = jnp.broadcast_to(yp[:, None, :], (c_dim, 48, nt))
            k3 = jnp.broadcast_to(k48[None, :, :], (c_dim, 48, nt))
            e = e + (yp3 * k3).reshape(c_dim * 48, nt)
            return d2, e

        carry = (d2t, e_init)
        for p in range(_PATCH):
            carry = step(p, carry)
        e_ref[0] = carry[1]

    return body


def _level_einsum(s, st, tt, ft, radius):
    B, Ns, _ = s.shape
    Nt = tt.shape[2]
    n_feat = 0 if ft is None else ft.shape[1]
    c_dim = (n_feat + 3) if n_feat else 4
    body = _make_level_body(radius, n_feat)
    in_specs = [
        pl.BlockSpec((1, Ns, 3), lambda b: (b, 0, 0)),
        pl.BlockSpec((1, 3, Ns), lambda b: (b, 0, 0)),
        pl.BlockSpec((1, 3, Nt), lambda b: (b, 0, 0)),
    ]
    args = [s, st, tt]
    if ft is not None:
        in_specs.append(pl.BlockSpec((1, n_feat, Ns), lambda b: (b, 0, 0)))
        args.append(ft)
    return pl.pallas_call(
        body,
        grid=(B,),
        in_specs=in_specs,
        out_specs=pl.BlockSpec((1, c_dim * 48, Nt), lambda b: (b, 0, 0)),
        out_shape=jax.ShapeDtypeStruct((B, c_dim * 48, Nt), jnp.float32),
        compiler_params=pltpu.CompilerParams(
            dimension_semantics=("parallel",)),
    )(*args)


def _mm_body(nk, wt_ref, xt_ref, b_ref, g_ref, be_ref, o_ref):
    k = pl.program_id(0)

    @pl.when(k == 0)
    def _():
        o_ref[...] = jnp.zeros_like(o_ref)

    o_ref[...] += jnp.dot(wt_ref[...], xt_ref[...],
                          preferred_element_type=jnp.float32)

    @pl.when(k == nk - 1)
    def _():
        acc = (o_ref[...] + b_ref[...]) * _BN_SCALE
        o_ref[...] = jax.nn.relu(g_ref[...] * acc + be_ref[...])


def _mlp_t(wt, xt, b, g, be, k_tile):
    U, K = wt.shape
    N = xt.shape[1]
    nk = K // k_tile
    assert K % k_tile == 0
    return pl.pallas_call(
        functools.partial(_mm_body, nk),
        grid=(nk,),
        in_specs=[
            pl.BlockSpec((U, k_tile), lambda k: (0, k)),
            pl.BlockSpec((k_tile, N), lambda k: (k, 0)),
            pl.BlockSpec((U, 1), lambda k: (0, 0)),
            pl.BlockSpec((U, 1), lambda k: (0, 0)),
            pl.BlockSpec((U, 1), lambda k: (0, 0)),
        ],
        out_specs=pl.BlockSpec((U, N), lambda k: (0, 0)),
        out_shape=jax.ShapeDtypeStruct((U, N), jnp.float32),
    )(wt, xt, b[:, None], g[:, None], be[:, None])


def _head_body(y_ref, w1_ref, b1_ref, g1_ref, be1_ref,
               w2_ref, b2_ref, g2_ref, be2_ref,
               wsm_ref, bsm_ref, o_ref):
    ym = jnp.max(y_ref[...], axis=1)
    h = jnp.dot(w1_ref[...], ym, preferred_element_type=jnp.float32)
    h = (h + b1_ref[...]) * _BN_SCALE
    h = jax.nn.relu(g1_ref[...] * h + be1_ref[...])
    h = jnp.dot(w2_ref[...], h, preferred_element_type=jnp.float32)
    h = (h + b2_ref[...]) * _BN_SCALE
    h = jax.nn.relu(g2_ref[...] * h + be2_ref[...])
    lg = jnp.dot(wsm_ref[...], h, preferred_element_type=jnp.float32)
    lg = lg + bsm_ref[...]
    m = jnp.max(lg, axis=0, keepdims=True)
    e = jnp.exp(lg - m)
    o_ref[...] = e / jnp.sum(e, axis=0, keepdims=True)


def kernel(x, W0, b0, g0, be0, W1, b1, g1, be1, W2, b2, g2, be2,
           Wfc1, bfc1, gfc1, befc1, Wfc2, bfc2, gfc2, befc2, Wsm, bsm):
    Ws = [W0, W1, W2]; bs = [b0, b1, b2]; gs = [g0, g1, g2]; bes = [be0, be1, be2]
    k_tiles = [192, 1664, 1792]
    k_pads = [192, 3328, 12544]
    x = _kdtree_indexing(x)
    points = [x]
    for i in range(3):
        points.append(_kd_pool(points[-1], _NUM_POINTS[i] // _NUM_POINTS[i + 1]))
    pt = [jnp.transpose(p, (0, 2, 1)) for p in points]
    B = x.shape[0]
    ft = None
    for i in range(3):
        Nt = _NUM_POINTS[i + 1]
        e = _level_einsum(points[i], pt[i], pt[i + 1], ft, _RADIUS[i])
        cd = e.shape[1]
        e_all = jnp.transpose(e, (1, 0, 2)).reshape(cd, B * Nt)
        kp = k_pads[i]
        wt = Ws[i].T
        if kp != cd:
            e_all = jnp.pad(e_all, ((0, kp - cd), (0, 0)))
            wt = jnp.pad(wt, ((0, 0), (0, kp - cd)))
        y = _mlp_t(wt, e_all, bs[i], gs[i], bes[i], k_tiles[i])
        U = y.shape[0]
        ft = jnp.transpose(y.reshape(U, B, Nt), (1, 0, 2))
    y3 = jnp.transpose(y.reshape(1024, B, 16), (0, 2, 1))
    out = pl.pallas_call(
        _head_body,
        out_shape=jax.ShapeDtypeStruct((Wsm.shape[1], B), jnp.float32),
    )(y3, Wfc1.T, bfc1[:, None], gfc1[:, None], befc1[:, None],
      Wfc2.T, bfc2[:, None], gfc2[:, None], befc2[:, None],
      Wsm.T, bsm[:, None])
    return out.T


# speedup vs baseline: 6.6636x; 1.1596x over previous
import functools

import jax
import jax.numpy as jnp
import numpy as np
from jax.experimental import pallas as pl
from jax.experimental.pallas import tpu as pltpu

_NUM_POINTS = [1024, 256, 64, 16]
_RADIUS = [0.2, 0.4, 0.8]
_PATCH = 32
_SHELLS = 3
_GAUSS_SCALE = 0.69314718056 * _SHELLS ** 2
_BN_EPS = 1e-3
_BN_SCALE = 1.0 / np.sqrt(1.0 + _BN_EPS)


def _kdtree_indexing(x):
    B, N, _ = x.shape
    depth = int(np.log2(N))
    y = x
    for lvl in range(depth):
        nb = 2 ** lvl
        blk = N // nb
        yb = y.reshape(B, nb, blk, 3)
        dim = lvl % 3
        cols = [yb[..., d] for d in range(3)]
        srt = jax.lax.sort([cols[dim]] + [cols[d] for d in range(3) if d != dim],
                           dimension=-1, num_keys=1)
        perm = list(range(3))
        others = [d for d in range(3) if d != dim]
        ordered = [None] * 3
        ordered[dim] = srt[0]
        ordered[others[0]] = srt[1]
        ordered[others[1]] = srt[2]
        y = jnp.stack(ordered, axis=-1).reshape(B, N, 3)
    return y


def _kd_pool(x, p):
    B, N, C = x.shape
    return x.reshape(B, N // p, p, C).mean(axis=2)


def _sh_rows(dx, dy, dz):
    one = jnp.ones_like(dx)
    return [0.28209479 * one,
            0.48860251 * dy, 0.48860251 * dz, 0.48860251 * dx,
            1.09254843 * dx * dy, 1.09254843 * dy * dz,
            0.31539157 * (3.0 * dz * dz - 1.0),
            1.09254843 * dx * dz, 0.54627421 * (dx * dx - dy * dy),
            0.59004359 * dy * (3.0 * dx * dx - dy * dy),
            2.89061144 * dx * dy * dz,
            0.45704579 * dy * (5.0 * dz * dz - 1.0),
            0.37317633 * dz * (5.0 * dz * dz - 3.0),
            0.45704579 * dx * (5.0 * dz * dz - 1.0),
            1.44530572 * dz * (dx * dx - dy * dy),
            0.59004359 * dx * (dx * dx - 3.0 * dy * dy)]


def _make_level_body(radius, n_feat):
    inv_r = 1.0 / radius

    def body(*refs):
        if n_feat:
            s_ref, st_ref, tt_ref, ft_ref, e_ref = refs
        else:
            s_ref, st_ref, tt_ref, e_ref = refs
        s = s_ref[0]
        st = st_ref[0]
        tt = tt_ref[0]
        if n_feat:
            gat = jnp.concatenate([st, ft_ref[0]], axis=0)
        else:
            gat = st
        nt = tt.shape[1]
        c_dim = (n_feat + 3) if n_feat else 4

        ssq = jnp.sum(s * s, axis=1, keepdims=True)
        tsq = jnp.sum(tt * tt, axis=0, keepdims=True)
        cross = jnp.dot(s, tt, preferred_element_type=jnp.float32)
        d2t = ssq - 2.0 * cross + tsq

        e_init = jnp.zeros((c_dim * 48, nt), jnp.float32)

        def step(_, carry):
            d2, e = carry
            m = jnp.min(d2, axis=0, keepdims=True)
            sel = d2 <= m
            d2 = jnp.where(sel, 1e30, d2)
            oh = sel.astype(jnp.float32)
            if n_feat:
                g = jnp.dot(gat, oh, preferred_element_type=jnp.float32)
                gc = g[0:3]
            else:
                gc = jnp.concatenate(
                    [jnp.sum(oh * s[:, r:r + 1], axis=0, keepdims=True)
                     for r in range(3)], axis=0)
            rel = (gc - tt) * inv_r
            dd = jnp.sum(rel * rel, axis=0, keepdims=True)
            dist = jnp.sqrt(jnp.maximum(dd, 1e-12))
            dirv = rel / jnp.maximum(dist, 1e-8)
            sh = jnp.concatenate(
                _sh_rows(dirv[0:1], dirv[1:2], dirv[2:3]), axis=0)
            in_r = (dist <= 1.0).astype(jnp.float32)
            shells = []
            for j in range(_SHELLS):
                cj = j / (_SHELLS - 1.0)
                gj = jnp.exp(-_GAUSS_SCALE * (dist - cj) ** 2) * in_r
                shells.append(gj * sh)
            k48 = jnp.concatenate(shells, axis=0)
            if n_feat:
                yp = jnp.concatenate(
                    [g[3:], gc[1:2], gc[2:3], gc[0:1]], axis=0)
            else:
                yp = jnp.concatenate(
                    [jnp.ones((1, nt), jnp.float32),
                     gc[1:2], gc[2:3], gc[0:1]], axis=0)
            yp3 = jnp.broadcast_to(yp[:, None, :], (c_dim, 48, nt))
            k3 = jnp.broadcast_to(k48[None, :, :], (c_dim, 48, nt))
            e = e + (yp3 * k3).reshape(c_dim * 48, nt)
            return d2, e

        carry = (d2t, e_init)
        for p in range(_PATCH):
            carry = step(p, carry)
        e_ref[0] = carry[1]

    return body


def _level_einsum(s, st, tt, ft, radius):
    B, Ns, _ = s.shape
    Nt = tt.shape[2]
    n_feat = 0 if ft is None else ft.shape[1]
    c_dim = (n_feat + 3) if n_feat else 4
    body = _make_level_body(radius, n_feat)
    in_specs = [
        pl.BlockSpec((1, Ns, 3), lambda b: (b, 0, 0)),
        pl.BlockSpec((1, 3, Ns), lambda b: (b, 0, 0)),
        pl.BlockSpec((1, 3, Nt), lambda b: (b, 0, 0)),
    ]
    args = [s, st, tt]
    if ft is not None:
        in_specs.append(pl.BlockSpec((1, n_feat, Ns), lambda b: (b, 0, 0)))
        args.append(ft)
    return pl.pallas_call(
        body,
        grid=(B,),
        in_specs=in_specs,
        out_specs=pl.BlockSpec((1, c_dim * 48, Nt), lambda b: (b, 0, 0)),
        out_shape=jax.ShapeDtypeStruct((B, c_dim * 48, Nt), jnp.float32),
        compiler_params=pltpu.CompilerParams(
            dimension_semantics=("parallel",)),
    )(*args)


def _mm_body(nk, wt_ref, xt_ref, b_ref, g_ref, be_ref, o_ref):
    k = pl.program_id(0)

    @pl.when(k == 0)
    def _():
        o_ref[...] = jnp.zeros_like(o_ref)

    o_ref[...] += jnp.dot(wt_ref[...], xt_ref[...],
                          preferred_element_type=jnp.float32)

    @pl.when(k == nk - 1)
    def _():
        acc = (o_ref[...] + b_ref[...]) * _BN_SCALE
        o_ref[...] = jax.nn.relu(g_ref[...] * acc + be_ref[...])


def _mlp_t(wt, xt, b, g, be, k_tile):
    U, K = wt.shape
    N = xt.shape[1]
    nk = K // k_tile
    assert K % k_tile == 0
    return pl.pallas_call(
        functools.partial(_mm_body, nk),
        grid=(nk,),
        in_specs=[
            pl.BlockSpec((U, k_tile), lambda k: (0, k)),
            pl.BlockSpec((k_tile, N), lambda k: (k, 0)),
            pl.BlockSpec((U, 1), lambda k: (0, 0)),
            pl.BlockSpec((U, 1), lambda k: (0, 0)),
            pl.BlockSpec((U, 1), lambda k: (0, 0)),
        ],
        out_specs=pl.BlockSpec((U, N), lambda k: (0, 0)),
        out_shape=jax.ShapeDtypeStruct((U, N), jnp.float32),
    )(wt, xt, b[:, None], g[:, None], be[:, None])


def _head_body(y_ref, w1_ref, b1_ref, g1_ref, be1_ref,
               w2_ref, b2_ref, g2_ref, be2_ref,
               wsm_ref, bsm_ref, o_ref):
    ym = jnp.max(y_ref[...], axis=1)
    h = jnp.dot(w1_ref[...], ym, preferred_element_type=jnp.float32)
    h = (h + b1_ref[...]) * _BN_SCALE
    h = jax.nn.relu(g1_ref[...] * h + be1_ref[...])
    h = jnp.dot(w2_ref[...], h, preferred_element_type=jnp.float32)
    h = (h + b2_ref[...]) * _BN_SCALE
    h = jax.nn.relu(g2_ref[...] * h + be2_ref[...])
    lg = jnp.dot(wsm_ref[...], h, preferred_element_type=jnp.float32)
    lg = lg + bsm_ref[...]
    m = jnp.max(lg, axis=0, keepdims=True)
    e = jnp.exp(lg - m)
    o_ref[...] = e / jnp.sum(e, axis=0, keepdims=True)


def kernel(x, W0, b0, g0, be0, W1, b1, g1, be1, W2, b2, g2, be2,
           Wfc1, bfc1, gfc1, befc1, Wfc2, bfc2, gfc2, befc2, Wsm, bsm):
    Ws = [W0, W1, W2]; bs = [b0, b1, b2]; gs = [g0, g1, g2]; bes = [be0, be1, be2]
    k_tiles = [192, 1664, 1792]
    k_pads = [192, 3328, 12544]
    x = _kdtree_indexing(x)
    points = [x]
    for i in range(3):
        points.append(_kd_pool(points[-1], _NUM_POINTS[i] // _NUM_POINTS[i + 1]))
    pt = [jnp.transpose(p, (0, 2, 1)) for p in points]
    B = x.shape[0]
    ft = None
    for i in range(3):
        Nt = _NUM_POINTS[i + 1]
        e = _level_einsum(points[i], pt[i], pt[i + 1], ft, _RADIUS[i])
        cd = e.shape[1]
        e_all = jnp.transpose(e, (1, 0, 2)).reshape(cd, B * Nt)
        kp = k_pads[i]
        wt = Ws[i].T
        if kp != cd:
            e_all = jnp.pad(e_all, ((0, kp - cd), (0, 0)))
            wt = jnp.pad(wt, ((0, 0), (0, kp - cd)))
        y = _mlp_t(wt, e_all, bs[i], gs[i], bes[i], k_tiles[i])
        U = y.shape[0]
        ft = jnp.transpose(y.reshape(U, B, Nt), (1, 0, 2))
    y3 = jnp.transpose(y.reshape(1024, B, 16), (0, 2, 1))
    out = pl.pallas_call(
        _head_body,
        out_shape=jax.ShapeDtypeStruct((Wsm.shape[1], B), jnp.float32),
    )(y3, Wfc1.T, bfc1[:, None], gfc1[:, None], befc1[:, None],
      Wfc2.T, bfc2[:, None], gfc2[:, None], befc2[:, None],
      Wsm.T, bsm[:, None])
    return out.T


# speedup vs baseline: 7.4803x; 1.1226x over previous
import functools

import jax
import jax.numpy as jnp
import numpy as np
from jax.experimental import pallas as pl
from jax.experimental.pallas import tpu as pltpu

_NUM_POINTS = [1024, 256, 64, 16]
_RADIUS = [0.2, 0.4, 0.8]
_PATCH = 32
_SHELLS = 3
_GAUSS_SCALE = 0.69314718056 * _SHELLS ** 2
_BN_EPS = 1e-3
_BN_SCALE = 1.0 / np.sqrt(1.0 + _BN_EPS)


def _kdtree_indexing(x):
    B, N, _ = x.shape
    depth = int(np.log2(N))
    y = x
    for lvl in range(depth):
        nb = 2 ** lvl
        blk = N // nb
        yb = y.reshape(B, nb, blk, 3)
        dim = lvl % 3
        cols = [yb[..., d] for d in range(3)]
        srt = jax.lax.sort([cols[dim]] + [cols[d] for d in range(3) if d != dim],
                           dimension=-1, num_keys=1)
        perm = list(range(3))
        others = [d for d in range(3) if d != dim]
        ordered = [None] * 3
        ordered[dim] = srt[0]
        ordered[others[0]] = srt[1]
        ordered[others[1]] = srt[2]
        y = jnp.stack(ordered, axis=-1).reshape(B, N, 3)
    return y


def _kd_pool(x, p):
    B, N, C = x.shape
    return x.reshape(B, N // p, p, C).mean(axis=2)


def _sh_rows(dx, dy, dz):
    one = jnp.ones_like(dx)
    return [0.28209479 * one,
            0.48860251 * dy, 0.48860251 * dz, 0.48860251 * dx,
            1.09254843 * dx * dy, 1.09254843 * dy * dz,
            0.31539157 * (3.0 * dz * dz - 1.0),
            1.09254843 * dx * dz, 0.54627421 * (dx * dx - dy * dy),
            0.59004359 * dy * (3.0 * dx * dx - dy * dy),
            2.89061144 * dx * dy * dz,
            0.45704579 * dy * (5.0 * dz * dz - 1.0),
            0.37317633 * dz * (5.0 * dz * dz - 3.0),
            0.45704579 * dx * (5.0 * dz * dz - 1.0),
            1.44530572 * dz * (dx * dx - dy * dy),
            0.59004359 * dx * (dx * dx - 3.0 * dy * dy)]


def _make_level_body(radius, n_feat):
    inv_r = 1.0 / radius

    def body(*refs):
        if n_feat:
            s_ref, st_ref, tt_ref, ft_ref, e_ref = refs
        else:
            s_ref, st_ref, tt_ref, e_ref = refs
        s = s_ref[0]
        st = st_ref[0]
        tt = tt_ref[0]
        if n_feat:
            gat = jnp.concatenate([st, ft_ref[0]], axis=0)
        else:
            gat = st
        nt = tt.shape[1]
        c_dim = (n_feat + 3) if n_feat else 4

        if not n_feat:
            st_hi = st.astype(jnp.bfloat16)
            st_lo = (st - st_hi.astype(jnp.float32)).astype(jnp.bfloat16)
        ssq = jnp.sum(s * s, axis=1, keepdims=True)
        tsq = jnp.sum(tt * tt, axis=0, keepdims=True)
        cross = jnp.dot(s, tt, preferred_element_type=jnp.float32)
        d2t = ssq - 2.0 * cross + tsq

        e_init = jnp.zeros((c_dim * 48, nt), jnp.float32)

        def step(_, carry):
            d2, e = carry
            m = jnp.min(d2, axis=0, keepdims=True)
            sel = d2 <= m
            d2 = jnp.where(sel, 1e30, d2)
            if n_feat:
                oh = sel.astype(jnp.float32)
                g = jnp.dot(gat, oh, preferred_element_type=jnp.float32)
                gc = g[0:3]
            else:
                ohb = sel.astype(jnp.bfloat16)
                gc = (jnp.dot(st_hi, ohb, preferred_element_type=jnp.float32)
                      + jnp.dot(st_lo, ohb, preferred_element_type=jnp.float32))
            rel = (gc - tt) * inv_r
            dd = jnp.sum(rel * rel, axis=0, keepdims=True)
            dist = jnp.sqrt(jnp.maximum(dd, 1e-12))
            dirv = rel / jnp.maximum(dist, 1e-8)
            sh = jnp.concatenate(
                _sh_rows(dirv[0:1], dirv[1:2], dirv[2:3]), axis=0)
            in_r = (dist <= 1.0).astype(jnp.float32)
            shells = []
            for j in range(_SHELLS):
                cj = j / (_SHELLS - 1.0)
                gj = jnp.exp(-_GAUSS_SCALE * (dist - cj) ** 2) * in_r
                shells.append(gj * sh)
            k48 = jnp.concatenate(shells, axis=0)
            if n_feat:
                yp = jnp.concatenate(
                    [g[3:], gc[1:2], gc[2:3], gc[0:1]], axis=0)
            else:
                yp = jnp.concatenate(
                    [jnp.ones((1, nt), jnp.float32),
                     gc[1:2], gc[2:3], gc[0:1]], axis=0)
            yp3 = jnp.broadcast_to(yp[:, None, :], (c_dim, 48, nt))
            k3 = jnp.broadcast_to(k48[None, :, :], (c_dim, 48, nt))
            e = e + (yp3 * k3).reshape(c_dim * 48, nt)
            return d2, e

        carry = (d2t, e_init)
        for p in range(_PATCH):
            carry = step(p, carry)
        e_ref[0] = carry[1]

    return body


def _level_einsum(s, st, tt, ft, radius):
    B, Ns, _ = s.shape
    Nt = tt.shape[2]
    n_feat = 0 if ft is None else ft.shape[1]
    c_dim = (n_feat + 3) if n_feat else 4
    body = _make_level_body(radius, n_feat)
    in_specs = [
        pl.BlockSpec((1, Ns, 3), lambda b: (b, 0, 0)),
        pl.BlockSpec((1, 3, Ns), lambda b: (b, 0, 0)),
        pl.BlockSpec((1, 3, Nt), lambda b: (b, 0, 0)),
    ]
    args = [s, st, tt]
    if ft is not None:
        in_specs.append(pl.BlockSpec((1, n_feat, Ns), lambda b: (b, 0, 0)))
        args.append(ft)
    return pl.pallas_call(
        body,
        grid=(B,),
        in_specs=in_specs,
        out_specs=pl.BlockSpec((1, c_dim * 48, Nt), lambda b: (b, 0, 0)),
        out_shape=jax.ShapeDtypeStruct((B, c_dim * 48, Nt), jnp.float32),
        compiler_params=pltpu.CompilerParams(
            dimension_semantics=("parallel",)),
    )(*args)


def _mm_body(nk, wt_ref, xt_ref, b_ref, g_ref, be_ref, o_ref):
    k = pl.program_id(0)

    @pl.when(k == 0)
    def _():
        o_ref[...] = jnp.zeros_like(o_ref)

    o_ref[...] += jnp.dot(wt_ref[...], xt_ref[...],
                          preferred_element_type=jnp.float32)

    @pl.when(k == nk - 1)
    def _():
        acc = (o_ref[...] + b_ref[...]) * _BN_SCALE
        o_ref[...] = jax.nn.relu(g_ref[...] * acc + be_ref[...])


def _mlp_t(wt, xt, b, g, be, k_tile):
    U, K = wt.shape
    N = xt.shape[1]
    nk = K // k_tile
    assert K % k_tile == 0
    return pl.pallas_call(
        functools.partial(_mm_body, nk),
        grid=(nk,),
        in_specs=[
            pl.BlockSpec((U, k_tile), lambda k: (0, k)),
            pl.BlockSpec((k_tile, N), lambda k: (k, 0)),
            pl.BlockSpec((U, 1), lambda k: (0, 0)),
            pl.BlockSpec((U, 1), lambda k: (0, 0)),
            pl.BlockSpec((U, 1), lambda k: (0, 0)),
        ],
        out_specs=pl.BlockSpec((U, N), lambda k: (0, 0)),
        out_shape=jax.ShapeDtypeStruct((U, N), jnp.float32),
    )(wt, xt, b[:, None], g[:, None], be[:, None])


def _head_body(y_ref, w1_ref, b1_ref, g1_ref, be1_ref,
               w2_ref, b2_ref, g2_ref, be2_ref,
               wsm_ref, bsm_ref, o_ref):
    ym = jnp.max(y_ref[...], axis=1)
    h = jnp.dot(w1_ref[...], ym, preferred_element_type=jnp.float32)
    h = (h + b1_ref[...]) * _BN_SCALE
    h = jax.nn.relu(g1_ref[...] * h + be1_ref[...])
    h = jnp.dot(w2_ref[...], h, preferred_element_type=jnp.float32)
    h = (h + b2_ref[...]) * _BN_SCALE
    h = jax.nn.relu(g2_ref[...] * h + be2_ref[...])
    lg = jnp.dot(wsm_ref[...], h, preferred_element_type=jnp.float32)
    lg = lg + bsm_ref[...]
    m = jnp.max(lg, axis=0, keepdims=True)
    e = jnp.exp(lg - m)
    o_ref[...] = e / jnp.sum(e, axis=0, keepdims=True)


def kernel(x, W0, b0, g0, be0, W1, b1, g1, be1, W2, b2, g2, be2,
           Wfc1, bfc1, gfc1, befc1, Wfc2, bfc2, gfc2, befc2, Wsm, bsm):
    Ws = [W0, W1, W2]; bs = [b0, b1, b2]; gs = [g0, g1, g2]; bes = [be0, be1, be2]
    k_tiles = [192, 1664, 1792]
    k_pads = [192, 3328, 12544]
    x = _kdtree_indexing(x)
    points = [x]
    for i in range(3):
        points.append(_kd_pool(points[-1], _NUM_POINTS[i] // _NUM_POINTS[i + 1]))
    pt = [jnp.transpose(p, (0, 2, 1)) for p in points]
    B = x.shape[0]
    ft = None
    for i in range(3):
        Nt = _NUM_POINTS[i + 1]
        e = _level_einsum(points[i], pt[i], pt[i + 1], ft, _RADIUS[i])
        cd = e.shape[1]
        e_all = jnp.transpose(e, (1, 0, 2)).reshape(cd, B * Nt)
        kp = k_pads[i]
        wt = Ws[i].T
        if kp != cd:
            e_all = jnp.pad(e_all, ((0, kp - cd), (0, 0)))
            wt = jnp.pad(wt, ((0, 0), (0, kp - cd)))
        y = _mlp_t(wt, e_all, bs[i], gs[i], bes[i], k_tiles[i])
        U = y.shape[0]
        ft = jnp.transpose(y.reshape(U, B, Nt), (1, 0, 2))
    y3 = jnp.transpose(y.reshape(1024, B, 16), (0, 2, 1))
    out = pl.pallas_call(
        _head_body,
        out_shape=jax.ShapeDtypeStruct((Wsm.shape[1], B), jnp.float32),
    )(y3, Wfc1.T, bfc1[:, None], gfc1[:, None], befc1[:, None],
      Wfc2.T, bfc2[:, None], gfc2[:, None], befc2[:, None],
      Wsm.T, bsm[:, None])
    return out.T


# speedup vs baseline: 13.3712x; 1.7875x over previous
import functools

import jax
import jax.numpy as jnp
import numpy as np
from jax.experimental import pallas as pl
from jax.experimental.pallas import tpu as pltpu

_NUM_POINTS = [1024, 256, 64, 16]
_RADIUS = [0.2, 0.4, 0.8]
_PATCH = 32
_SHELLS = 3
_GAUSS_SCALE = 0.69314718056 * _SHELLS ** 2
_BN_EPS = 1e-3
_BN_SCALE = 1.0 / np.sqrt(1.0 + _BN_EPS)


def _kdtree_indexing(x):
    B, N, _ = x.shape
    depth = int(np.log2(N))
    y = x
    for lvl in range(depth):
        nb = 2 ** lvl
        blk = N // nb
        yb = y.reshape(B, nb, blk, 3)
        dim = lvl % 3
        cols = [yb[..., d] for d in range(3)]
        srt = jax.lax.sort([cols[dim]] + [cols[d] for d in range(3) if d != dim],
                           dimension=-1, num_keys=1)
        perm = list(range(3))
        others = [d for d in range(3) if d != dim]
        ordered = [None] * 3
        ordered[dim] = srt[0]
        ordered[others[0]] = srt[1]
        ordered[others[1]] = srt[2]
        y = jnp.stack(ordered, axis=-1).reshape(B, N, 3)
    return y


def _kd_pool(x, p):
    B, N, C = x.shape
    return x.reshape(B, N // p, p, C).mean(axis=2)


def _sh_rows(dx, dy, dz):
    one = jnp.ones_like(dx)
    return [0.28209479 * one,
            0.48860251 * dy, 0.48860251 * dz, 0.48860251 * dx,
            1.09254843 * dx * dy, 1.09254843 * dy * dz,
            0.31539157 * (3.0 * dz * dz - 1.0),
            1.09254843 * dx * dz, 0.54627421 * (dx * dx - dy * dy),
            0.59004359 * dy * (3.0 * dx * dx - dy * dy),
            2.89061144 * dx * dy * dz,
            0.45704579 * dy * (5.0 * dz * dz - 1.0),
            0.37317633 * dz * (5.0 * dz * dz - 3.0),
            0.45704579 * dx * (5.0 * dz * dz - 1.0),
            1.44530572 * dz * (dx * dx - dy * dy),
            0.59004359 * dx * (dx * dx - 3.0 * dy * dy)]


def _make_level_body(radius, n_feat, ns_item, nt_item):
    inv_r = 1.0 / radius

    def body(*refs):
        if n_feat:
            s_ref, st_ref, tt_ref, ft_ref, e_ref = refs
        else:
            s_ref, st_ref, tt_ref, e_ref = refs
        s = s_ref[0]
        st = st_ref[0]
        tt = tt_ref[0]
        if n_feat:
            gat = jnp.concatenate([st, ft_ref[0]], axis=0)
        else:
            gat = st
        nt = tt.shape[1]
        c_dim = (n_feat + 3) if n_feat else 4

        if not n_feat:
            st_hi = st.astype(jnp.bfloat16)
            st_lo = (st - st_hi.astype(jnp.float32)).astype(jnp.bfloat16)
        ssq = jnp.sum(s * s, axis=1, keepdims=True)
        tsq = jnp.sum(tt * tt, axis=0, keepdims=True)
        cross = jnp.dot(s, tt, preferred_element_type=jnp.float32)
        d2t = ssq - 2.0 * cross + tsq
        ns_tot = s.shape[0]
        if ns_tot != ns_item:
            item_r = jax.lax.broadcasted_iota(
                jnp.int32, (ns_tot, nt), 0) // ns_item
            item_c = jax.lax.broadcasted_iota(
                jnp.int32, (ns_tot, nt), 1) // nt_item
            d2t = jnp.where(item_r == item_c, d2t, 1e30)

        e_init = jnp.zeros((c_dim * 48, nt), jnp.float32)

        def step(_, carry):
            d2, e = carry
            m = jnp.min(d2, axis=0, keepdims=True)
            sel = d2 <= m
            d2 = jnp.where(sel, 1e30, d2)
            if n_feat:
                oh = sel.astype(jnp.float32)
                g = jnp.dot(gat, oh, preferred_element_type=jnp.float32)
                gc = g[0:3]
            else:
                ohb = sel.astype(jnp.bfloat16)
                gc = (jnp.dot(st_hi, ohb, preferred_element_type=jnp.float32)
                      + jnp.dot(st_lo, ohb, preferred_element_type=jnp.float32))
            rel = (gc - tt) * inv_r
            dd = jnp.sum(rel * rel, axis=0, keepdims=True)
            dist = jnp.sqrt(jnp.maximum(dd, 1e-12))
            dirv = rel / jnp.maximum(dist, 1e-8)
            sh = jnp.concatenate(
                _sh_rows(dirv[0:1], dirv[1:2], dirv[2:3]), axis=0)
            in_r = (dist <= 1.0).astype(jnp.float32)
            shells = []
            for j in range(_SHELLS):
                cj = j / (_SHELLS - 1.0)
                gj = jnp.exp(-_GAUSS_SCALE * (dist - cj) ** 2) * in_r
                shells.append(gj * sh)
            k48 = jnp.concatenate(shells, axis=0)
            if n_feat:
                yp = jnp.concatenate(
                    [g[3:], gc[1:2], gc[2:3], gc[0:1]], axis=0)
            else:
                yp = jnp.concatenate(
                    [jnp.ones((1, nt), jnp.float32),
                     gc[1:2], gc[2:3], gc[0:1]], axis=0)
            yp3 = jnp.broadcast_to(yp[:, None, :], (c_dim, 48, nt))
            k3 = jnp.broadcast_to(k48[None, :, :], (c_dim, 48, nt))
            e = e + (yp3 * k3).reshape(c_dim * 48, nt)
            return d2, e

        carry = (d2t, e_init)
        for p in range(_PATCH):
            carry = step(p, carry)
        e_ref[0] = carry[1]

    return body


def _level_einsum(s, st, tt, ft, radius, g_merge):
    B, Ns, _ = s.shape
    Nt = tt.shape[2]
    n_feat = 0 if ft is None else ft.shape[1]
    c_dim = (n_feat + 3) if n_feat else 4
    body = _make_level_body(radius, n_feat, Ns // g_merge, Nt // g_merge)
    in_specs = [
        pl.BlockSpec((1, Ns, 3), lambda b: (b, 0, 0)),
        pl.BlockSpec((1, 3, Ns), lambda b: (b, 0, 0)),
        pl.BlockSpec((1, 3, Nt), lambda b: (b, 0, 0)),
    ]
    args = [s, st, tt]
    if ft is not None:
        in_specs.append(pl.BlockSpec((1, n_feat, Ns), lambda b: (b, 0, 0)))
        args.append(ft)
    return pl.pallas_call(
        body,
        grid=(B,),
        in_specs=in_specs,
        out_specs=pl.BlockSpec((1, c_dim * 48, Nt), lambda b: (b, 0, 0)),
        out_shape=jax.ShapeDtypeStruct((B, c_dim * 48, Nt), jnp.float32),
        compiler_params=pltpu.CompilerParams(
            dimension_semantics=("parallel",)),
    )(*args)


def _mm_body(nk, wt_ref, xt_ref, b_ref, g_ref, be_ref, o_ref):
    k = pl.program_id(0)

    @pl.when(k == 0)
    def _():
        o_ref[...] = jnp.zeros_like(o_ref)

    o_ref[...] += jnp.dot(wt_ref[...], xt_ref[...],
                          preferred_element_type=jnp.float32)

    @pl.when(k == nk - 1)
    def _():
        acc = (o_ref[...] + b_ref[...]) * _BN_SCALE
        o_ref[...] = jax.nn.relu(g_ref[...] * acc + be_ref[...])


def _mlp_t(wt, xt, b, g, be, k_tile):
    U, K = wt.shape
    N = xt.shape[1]
    nk = K // k_tile
    assert K % k_tile == 0
    return pl.pallas_call(
        functools.partial(_mm_body, nk),
        grid=(nk,),
        in_specs=[
            pl.BlockSpec((U, k_tile), lambda k: (0, k)),
            pl.BlockSpec((k_tile, N), lambda k: (k, 0)),
            pl.BlockSpec((U, 1), lambda k: (0, 0)),
            pl.BlockSpec((U, 1), lambda k: (0, 0)),
            pl.BlockSpec((U, 1), lambda k: (0, 0)),
        ],
        out_specs=pl.BlockSpec((U, N), lambda k: (0, 0)),
        out_shape=jax.ShapeDtypeStruct((U, N), jnp.float32),
    )(wt, xt, b[:, None], g[:, None], be[:, None])


def _head_body(y_ref, w1_ref, b1_ref, g1_ref, be1_ref,
               w2_ref, b2_ref, g2_ref, be2_ref,
               wsm_ref, bsm_ref, o_ref):
    ym = jnp.max(y_ref[...], axis=1)
    h = jnp.dot(w1_ref[...], ym, preferred_element_type=jnp.float32)
    h = (h + b1_ref[...]) * _BN_SCALE
    h = jax.nn.relu(g1_ref[...] * h + be1_ref[...])
    h = jnp.dot(w2_ref[...], h, preferred_element_type=jnp.float32)
    h = (h + b2_ref[...]) * _BN_SCALE
    h = jax.nn.relu(g2_ref[...] * h + be2_ref[...])
    lg = jnp.dot(wsm_ref[...], h, preferred_element_type=jnp.float32)
    lg = lg + bsm_ref[...]
    m = jnp.max(lg, axis=0, keepdims=True)
    e = jnp.exp(lg - m)
    o_ref[...] = e / jnp.sum(e, axis=0, keepdims=True)


def kernel(x, W0, b0, g0, be0, W1, b1, g1, be1, W2, b2, g2, be2,
           Wfc1, bfc1, gfc1, befc1, Wfc2, bfc2, gfc2, befc2, Wsm, bsm):
    Ws = [W0, W1, W2]; bs = [b0, b1, b2]; gs = [g0, g1, g2]; bes = [be0, be1, be2]
    k_tiles = [192, 1664, 1792]
    k_pads = [192, 3328, 12544]
    x = _kdtree_indexing(x)
    points = [x]
    for i in range(3):
        points.append(_kd_pool(points[-1], _NUM_POINTS[i] // _NUM_POINTS[i + 1]))
    B = x.shape[0]
    g_merges = [1, 2, 8]
    ft = None
    for i in range(3):
        Ns, Nt = _NUM_POINTS[i], _NUM_POINTS[i + 1]
        G = min(g_merges[i], B)
        s_m = points[i].reshape(B // G, G * Ns, 3)
        st_m = jnp.transpose(s_m, (0, 2, 1))
        t_m = points[i + 1].reshape(B // G, G * Nt, 3)
        tt_m = jnp.transpose(t_m, (0, 2, 1))
        if ft is not None:
            U = ft.shape[0]
            ft_m = jnp.transpose(ft.reshape(U, B // G, G * Ns), (1, 0, 2))
        else:
            ft_m = None
        e = _level_einsum(s_m, st_m, tt_m, ft_m, _RADIUS[i], G)
        cd = e.shape[1]
        e_all = jnp.transpose(e, (1, 0, 2)).reshape(cd, B * Nt)
        kp = k_pads[i]
        wt = Ws[i].T
        if kp != cd:
            e_all = jnp.pad(e_all, ((0, kp - cd), (0, 0)))
            wt = jnp.pad(wt, ((0, 0), (0, kp - cd)))
        y = _mlp_t(wt, e_all, bs[i], gs[i], bes[i], k_tiles[i])
        ft = y
    y3 = jnp.transpose(y.reshape(1024, B, 16), (0, 2, 1))
    out = pl.pallas_call(
        _head_body,
        out_shape=jax.ShapeDtypeStruct((Wsm.shape[1], B), jnp.float32),
    )(y3, Wfc1.T, bfc1[:, None], gfc1[:, None], befc1[:, None],
      Wfc2.T, bfc2[:, None], gfc2[:, None], befc2[:, None],
      Wsm.T, bsm[:, None])
    return out.T
